# Initial kernel scaffold; baseline (speedup 1.0000x reference)
import jax
import jax.numpy as jnp
from jax.experimental import pallas as pl


def kernel(x, pe, edge_index, edge_attr, batch, params):
    raise NotImplementedError("write your pallas kernel here")


# speedup vs baseline: 1.1123x; 1.1123x over previous
import functools
import math

import jax
import jax.numpy as jnp
from jax.experimental import pallas as pl
from jax.experimental.pallas import tpu as pltpu

_H = 4
_B = 20
_BQ = 256
_BK = 256
_PAD_BATCH = jnp.int32(1 << 20)


def _pad_rows(a, mult):
    n = a.shape[0]
    npad = -(-n // mult) * mult
    if npad == n:
        return a
    pad = [(0, npad - n)] + [(0, 0)] * (a.ndim - 1)
    return jnp.pad(a, pad)


def _linear_kernel(x_ref, w_ref, b_ref, o_ref, *, relu):
    y = jax.lax.dot_general(x_ref[...], w_ref[...], (((1,), (0,)), ((), ())),
                            preferred_element_type=jnp.float32)
    y = y + b_ref[...]
    if relu:
        y = jnp.maximum(y, 0.0)
    o_ref[...] = y


def _linear(x, w, b, relu=False, bm=256):
    n, k = x.shape
    kout = w.shape[1]
    xp = _pad_rows(x, bm)
    npad = xp.shape[0]
    out = pl.pallas_call(
        functools.partial(_linear_kernel, relu=relu),
        grid=(npad // bm,),
        in_specs=[
            pl.BlockSpec((bm, k), lambda i: (i, 0)),
            pl.BlockSpec((k, kout), lambda i: (0, 0)),
            pl.BlockSpec((1, kout), lambda i: (0, 0)),
        ],
        out_specs=pl.BlockSpec((bm, kout), lambda i: (i, 0)),
        out_shape=jax.ShapeDtypeStruct((npad, kout), jnp.float32),
    )(xp, w, b.reshape(1, -1))
    return out[:n]


def _bn_kernel(x_ref, g_ref, b_ref, o_ref, *, n_valid):
    x = x_ref[...]
    rows = jax.lax.broadcasted_iota(jnp.int32, x.shape, 0)
    valid = rows < n_valid
    xm = jnp.where(valid, x, 0.0)
    mu = jnp.sum(xm, axis=0, keepdims=True) / n_valid
    d = jnp.where(valid, x - mu, 0.0)
    var = jnp.sum(d * d, axis=0, keepdims=True) / n_valid
    o_ref[...] = (x - mu) / jnp.sqrt(var + 1e-5) * g_ref[...] + b_ref[...]


def _bn(x, g, b):
    n, c = x.shape
    xp = _pad_rows(x, 256)
    out = pl.pallas_call(
        functools.partial(_bn_kernel, n_valid=n),
        in_specs=[pl.BlockSpec(xp.shape, lambda: (0, 0)),
                  pl.BlockSpec((1, c), lambda: (0, 0)),
                  pl.BlockSpec((1, c), lambda: (0, 0))],
        out_specs=pl.BlockSpec(xp.shape, lambda: (0, 0)),
        out_shape=jax.ShapeDtypeStruct(xp.shape, jnp.float32),
    )(xp, g.reshape(1, -1), b.reshape(1, -1))
    return out[:n]


def _attn_kernel(kblo_ref, kbhi_ref, q_ref, k_ref, v_ref, bcol_ref, brow_ref,
                 o_ref, *, scale, bk):
    qi = pl.program_id(1)
    lo = kblo_ref[qi]
    hi = kbhi_ref[qi]
    q = q_ref[...]
    bq = bcol_ref[...]

    def body(kb, carry):
        acc, m, l = carry
        kblk = k_ref[pl.ds(kb * bk, bk), :]
        vblk = v_ref[pl.ds(kb * bk, bk), :]
        bkey = brow_ref[:, pl.ds(kb * bk, bk)]
        s = jax.lax.dot_general(q, kblk, (((1,), (1,)), ((), ())),
                                preferred_element_type=jnp.float32) * scale
        mask = bq == bkey
        s = jnp.where(mask, s, -1e9)
        m_new = jnp.maximum(m, jnp.max(s, axis=1, keepdims=True))
        p = jnp.where(mask, jnp.exp(s - m_new), 0.0)
        alpha = jnp.exp(m - m_new)
        l_new = l * alpha + jnp.sum(p, axis=1, keepdims=True)
        acc_new = acc * alpha + jax.lax.dot_general(
            p, vblk, (((1,), (0,)), ((), ())), preferred_element_type=jnp.float32)
        return acc_new, m_new, l_new

    init = (jnp.zeros(q.shape, jnp.float32),
            jnp.full((q.shape[0], 1), -1e9, jnp.float32),
            jnp.zeros((q.shape[0], 1), jnp.float32))
    acc, m, l = jax.lax.fori_loop(lo, hi, body, init)
    l = jnp.where(l == 0.0, 1.0, l)
    o_ref[...] = acc / l


def _attention(q, k, v, batch_pad):
    h, npad, dh = q.shape
    num_qb = npad // _BQ
    scale = 1.0 / math.sqrt(dh)

    qstarts = jnp.arange(num_qb) * _BQ
    b_lo = batch_pad[qstarts]
    b_hi = batch_pad[qstarts + _BQ - 1]
    lo_idx = jnp.searchsorted(batch_pad, b_lo, side='left')
    hi_idx = jnp.searchsorted(batch_pad, b_hi, side='right')
    kb_lo = (lo_idx // _BK).astype(jnp.int32)
    kb_hi = ((hi_idx + _BK - 1) // _BK).astype(jnp.int32)

    bcol = batch_pad.reshape(-1, 1)
    brow = batch_pad.reshape(1, -1)

    grid_spec = pltpu.PrefetchScalarGridSpec(
        num_scalar_prefetch=2,
        grid=(h, num_qb),
        in_specs=[
            pl.BlockSpec((None, _BQ, dh), lambda hh, qi, lo, hi: (hh, qi, 0)),
            pl.BlockSpec((None, npad, dh), lambda hh, qi, lo, hi: (hh, 0, 0)),
            pl.BlockSpec((None, npad, dh), lambda hh, qi, lo, hi: (hh, 0, 0)),
            pl.BlockSpec((_BQ, 1), lambda hh, qi, lo, hi: (qi, 0)),
            pl.BlockSpec((1, npad), lambda hh, qi, lo, hi: (0, 0)),
        ],
        out_specs=pl.BlockSpec((None, _BQ, dh), lambda hh, qi, lo, hi: (hh, qi, 0)),
    )
    return pl.pallas_call(
        functools.partial(_attn_kernel, scale=scale, bk=_BK),
        grid_spec=grid_spec,
        out_shape=jax.ShapeDtypeStruct((h, npad, dh), jnp.float32),
    )(kb_lo, kb_hi, q, k, v, bcol, brow)


def kernel(x, pe, edge_index, edge_attr, batch, params):
    n, c = x.shape
    dh = c // _H
    src = edge_index[0]
    dst = edge_index[1]
    batch = batch.astype(jnp.int32)
    batch_pad = _pad_rows(batch, _BQ)
    npad = batch_pad.shape[0]
    if npad != n:
        batch_pad = batch_pad.at[n:].set(_PAD_BATCH)

    pe_mu = jnp.mean(pe, axis=0)
    pe_var = jnp.var(pe, axis=0)
    x_pe = (pe - pe_mu) / jnp.sqrt(pe_var + 1e-5) * params['pe_norm_g'] + params['pe_norm_b']
    pe_walk = pe.shape[1]
    c_node = params['node_emb_w'].shape[1]
    w_emb = jnp.zeros((c + pe_walk, c), jnp.float32)
    w_emb = w_emb.at[:c, :c_node].set(params['node_emb_w'])
    w_emb = w_emb.at[c:, c_node:].set(params['pe_lin_w'])
    b_emb = jnp.concatenate([params['node_emb_b'], params['pe_lin_b']])
    x0 = _linear(jnp.concatenate([x, x_pe], axis=1), w_emb, b_emb)

    ea = _linear(edge_attr, params['edge_emb_w'], params['edge_emb_b'])

    for lp in params['layers']:
        msg = jnp.maximum(x0[src] + ea, 0.0)
        aggr = jax.ops.segment_sum(msg, dst, num_segments=n)
        hh = x0 + aggr
        hh = _linear(hh, lp['gine_w1'], lp['gine_b1'], relu=True)
        hh = _linear(hh, lp['gine_w2'], lp['gine_b2'])
        h1 = _bn(hh + x0, lp['n1_g'], lp['n1_b'])

        w_qkv = jnp.concatenate([lp['wq'], lp['wk'], lp['wv']], axis=1)
        b_qkv = jnp.concatenate([lp['bq'], lp['bk'], lp['bv']])
        qkv = _pad_rows(_linear(x0, w_qkv, b_qkv), _BQ)
        q = qkv[:, :c].reshape(npad, _H, dh).transpose(1, 0, 2)
        k = qkv[:, c:2 * c].reshape(npad, _H, dh).transpose(1, 0, 2)
        v = qkv[:, 2 * c:].reshape(npad, _H, dh).transpose(1, 0, 2)
        o = _attention(q, k, v, batch_pad)
        o = o.transpose(1, 0, 2).reshape(npad, c)[:n]
        o = _linear(o, lp['wo'], lp['bo'])
        h2 = _bn(o + x0, lp['n2_g'], lp['n2_b'])

        out = h1 + h2
        ff = _linear(out, lp['ff_w1'], lp['ff_b1'], relu=True)
        ff = _linear(ff, lp['ff_w2'], lp['ff_b2'])
        x0 = _bn(out + ff, lp['n3_g'], lp['n3_b'])

    pooled = jax.ops.segment_sum(x0, batch, num_segments=_B)
    y = _linear(pooled, params['mlp_w1'], params['mlp_b1'], relu=True, bm=32)
    y = _linear(y, params['mlp_w2'], params['mlp_b2'], relu=True, bm=32)
    return _linear(y, params['mlp_w3'], params['mlp_b3'], bm=32)


# speedup vs baseline: 1.2077x; 1.0857x over previous
import functools
import math

import jax
import jax.numpy as jnp
from jax.experimental import pallas as pl
from jax.experimental.pallas import tpu as pltpu

_H = 4
_B = 20
_BQ = 256
_BK = 256
_PAD_BATCH = jnp.int32(1 << 20)


def _pad_rows(a, mult):
    n = a.shape[0]
    npad = -(-n // mult) * mult
    if npad == n:
        return a
    pad = [(0, npad - n)] + [(0, 0)] * (a.ndim - 1)
    return jnp.pad(a, pad)


def _linear_kernel(x_ref, w_ref, b_ref, o_ref, *, relu):
    y = jax.lax.dot_general(x_ref[...], w_ref[...], (((1,), (0,)), ((), ())),
                            preferred_element_type=jnp.float32)
    y = y + b_ref[...]
    if relu:
        y = jnp.maximum(y, 0.0)
    o_ref[...] = y


def _linear(x, w, b, relu=False, bm=256):
    n, k = x.shape
    kout = w.shape[1]
    xp = _pad_rows(x, bm)
    npad = xp.shape[0]
    out = pl.pallas_call(
        functools.partial(_linear_kernel, relu=relu),
        grid=(npad // bm,),
        in_specs=[
            pl.BlockSpec((bm, k), lambda i: (i, 0)),
            pl.BlockSpec((k, kout), lambda i: (0, 0)),
            pl.BlockSpec((1, kout), lambda i: (0, 0)),
        ],
        out_specs=pl.BlockSpec((bm, kout), lambda i: (i, 0)),
        out_shape=jax.ShapeDtypeStruct((npad, kout), jnp.float32),
    )(xp, w, b.reshape(1, -1))
    return out[:n]


def _bn_kernel(x_ref, g_ref, b_ref, o_ref, *, n_valid):
    x = x_ref[...]
    rows = jax.lax.broadcasted_iota(jnp.int32, x.shape, 0)
    valid = rows < n_valid
    xm = jnp.where(valid, x, 0.0)
    mu = jnp.sum(xm, axis=0, keepdims=True) / n_valid
    d = jnp.where(valid, x - mu, 0.0)
    var = jnp.sum(d * d, axis=0, keepdims=True) / n_valid
    o_ref[...] = (x - mu) / jnp.sqrt(var + 1e-5) * g_ref[...] + b_ref[...]


def _bn(x, g, b):
    n, c = x.shape
    xp = _pad_rows(x, 256)
    out = pl.pallas_call(
        functools.partial(_bn_kernel, n_valid=n),
        in_specs=[pl.BlockSpec(xp.shape, lambda: (0, 0)),
                  pl.BlockSpec((1, c), lambda: (0, 0)),
                  pl.BlockSpec((1, c), lambda: (0, 0))],
        out_specs=pl.BlockSpec(xp.shape, lambda: (0, 0)),
        out_shape=jax.ShapeDtypeStruct(xp.shape, jnp.float32),
    )(xp, g.reshape(1, -1), b.reshape(1, -1))
    return out[:n]


_EB = 256
_NB = 256


def _segsum_kernel(wlo_ref, whi_ref, m_ref, dst_ref, o_ref):
    i = pl.program_id(0)

    @pl.when(i == 0)
    def _zero():
        o_ref[...] = jnp.zeros_like(o_ref)

    m = m_ref[...]
    d = dst_ref[...]
    lo = wlo_ref[i]
    hi = whi_ref[i]

    def body(j, carry):
        node_base = j * _NB
        rowids = node_base + jax.lax.broadcasted_iota(jnp.int32, (_NB, _EB), 0)
        onehot = jnp.where(rowids == d, 1.0, 0.0)
        contrib = jax.lax.dot_general(onehot, m, (((1,), (0,)), ((), ())),
                                      preferred_element_type=jnp.float32)
        o_ref[pl.ds(node_base, _NB), :] = o_ref[pl.ds(node_base, _NB), :] + contrib
        return carry

    jax.lax.fori_loop(lo, hi + 1, body, 0)


def _segsum_sorted(m, dst_sorted, n):
    e, c = m.shape
    mp = _pad_rows(m, _EB)
    epad = mp.shape[0]
    if epad != e:
        dst_sorted = jnp.concatenate(
            [dst_sorted, jnp.broadcast_to(dst_sorted[-1], (epad - e,))])
    npad = -(-n // _NB) * _NB
    num_eb = epad // _EB
    starts = jnp.arange(num_eb) * _EB
    wlo = (dst_sorted[starts] // _NB).astype(jnp.int32)
    whi = (dst_sorted[starts + _EB - 1] // _NB).astype(jnp.int32)

    grid_spec = pltpu.PrefetchScalarGridSpec(
        num_scalar_prefetch=2,
        grid=(num_eb,),
        in_specs=[
            pl.BlockSpec((_EB, c), lambda i, lo, hi: (i, 0)),
            pl.BlockSpec((1, _EB), lambda i, lo, hi: (0, i)),
        ],
        out_specs=pl.BlockSpec((npad, c), lambda i, lo, hi: (0, 0)),
    )
    out = pl.pallas_call(
        _segsum_kernel,
        grid_spec=grid_spec,
        out_shape=jax.ShapeDtypeStruct((npad, c), jnp.float32),
    )(wlo, whi, mp, dst_sorted.reshape(1, -1).astype(jnp.int32))
    return out[:n]


def _attn_kernel(kblo_ref, kbhi_ref, q_ref, k_ref, v_ref, bcol_ref, brow_ref,
                 o_ref, *, scale, bk):
    qi = pl.program_id(1)
    lo = kblo_ref[qi]
    hi = kbhi_ref[qi]
    q = q_ref[...]
    bq = bcol_ref[...]

    def body(kb, carry):
        acc, m, l = carry
        kblk = k_ref[pl.ds(kb * bk, bk), :]
        vblk = v_ref[pl.ds(kb * bk, bk), :]
        bkey = brow_ref[:, pl.ds(kb * bk, bk)]
        s = jax.lax.dot_general(q, kblk, (((1,), (1,)), ((), ())),
                                preferred_element_type=jnp.float32) * scale
        mask = bq == bkey
        s = jnp.where(mask, s, -1e9)
        m_new = jnp.maximum(m, jnp.max(s, axis=1, keepdims=True))
        p = jnp.where(mask, jnp.exp(s - m_new), 0.0)
        alpha = jnp.exp(m - m_new)
        l_new = l * alpha + jnp.sum(p, axis=1, keepdims=True)
        acc_new = acc * alpha + jax.lax.dot_general(
            p, vblk, (((1,), (0,)), ((), ())), preferred_element_type=jnp.float32)
        return acc_new, m_new, l_new

    init = (jnp.zeros(q.shape, jnp.float32),
            jnp.full((q.shape[0], 1), -1e9, jnp.float32),
            jnp.zeros((q.shape[0], 1), jnp.float32))
    acc, m, l = jax.lax.fori_loop(lo, hi, body, init)
    l = jnp.where(l == 0.0, 1.0, l)
    o_ref[...] = acc / l


def _attention(q, k, v, batch_pad):
    h, npad, dh = q.shape
    num_qb = npad // _BQ
    scale = 1.0 / math.sqrt(dh)

    qstarts = jnp.arange(num_qb) * _BQ
    b_lo = batch_pad[qstarts]
    b_hi = batch_pad[qstarts + _BQ - 1]
    lo_idx = jnp.searchsorted(batch_pad, b_lo, side='left')
    hi_idx = jnp.searchsorted(batch_pad, b_hi, side='right')
    kb_lo = (lo_idx // _BK).astype(jnp.int32)
    kb_hi = ((hi_idx + _BK - 1) // _BK).astype(jnp.int32)

    bcol = batch_pad.reshape(-1, 1)
    brow = batch_pad.reshape(1, -1)

    grid_spec = pltpu.PrefetchScalarGridSpec(
        num_scalar_prefetch=2,
        grid=(h, num_qb),
        in_specs=[
            pl.BlockSpec((None, _BQ, dh), lambda hh, qi, lo, hi: (hh, qi, 0)),
            pl.BlockSpec((None, npad, dh), lambda hh, qi, lo, hi: (hh, 0, 0)),
            pl.BlockSpec((None, npad, dh), lambda hh, qi, lo, hi: (hh, 0, 0)),
            pl.BlockSpec((_BQ, 1), lambda hh, qi, lo, hi: (qi, 0)),
            pl.BlockSpec((1, npad), lambda hh, qi, lo, hi: (0, 0)),
        ],
        out_specs=pl.BlockSpec((None, _BQ, dh), lambda hh, qi, lo, hi: (hh, qi, 0)),
    )
    return pl.pallas_call(
        functools.partial(_attn_kernel, scale=scale, bk=_BK),
        grid_spec=grid_spec,
        out_shape=jax.ShapeDtypeStruct((h, npad, dh), jnp.float32),
    )(kb_lo, kb_hi, q, k, v, bcol, brow)


def kernel(x, pe, edge_index, edge_attr, batch, params):
    n, c = x.shape
    dh = c // _H
    src = edge_index[0]
    dst = edge_index[1]
    batch = batch.astype(jnp.int32)
    batch_pad = _pad_rows(batch, _BQ)
    npad = batch_pad.shape[0]
    if npad != n:
        batch_pad = batch_pad.at[n:].set(_PAD_BATCH)

    pe_mu = jnp.mean(pe, axis=0)
    pe_var = jnp.var(pe, axis=0)
    x_pe = (pe - pe_mu) / jnp.sqrt(pe_var + 1e-5) * params['pe_norm_g'] + params['pe_norm_b']
    pe_walk = pe.shape[1]
    c_node = params['node_emb_w'].shape[1]
    w_emb = jnp.zeros((c + pe_walk, c), jnp.float32)
    w_emb = w_emb.at[:c, :c_node].set(params['node_emb_w'])
    w_emb = w_emb.at[c:, c_node:].set(params['pe_lin_w'])
    b_emb = jnp.concatenate([params['node_emb_b'], params['pe_lin_b']])
    x0 = _linear(jnp.concatenate([x, x_pe], axis=1), w_emb, b_emb)

    ea = _linear(edge_attr, params['edge_emb_w'], params['edge_emb_b'])

    perm = jnp.argsort(dst)
    src_s = src[perm]
    dst_s = dst[perm]
    ea_s = ea[perm]

    for lp in params['layers']:
        msg = jnp.maximum(x0[src_s] + ea_s, 0.0)
        aggr = _segsum_sorted(msg, dst_s, n)
        hh = x0 + aggr
        hh = _linear(hh, lp['gine_w1'], lp['gine_b1'], relu=True)
        hh = _linear(hh, lp['gine_w2'], lp['gine_b2'])
        h1 = _bn(hh + x0, lp['n1_g'], lp['n1_b'])

        w_qkv = jnp.concatenate([lp['wq'], lp['wk'], lp['wv']], axis=1)
        b_qkv = jnp.concatenate([lp['bq'], lp['bk'], lp['bv']])
        qkv = _pad_rows(_linear(x0, w_qkv, b_qkv), _BQ)
        q = qkv[:, :c].reshape(npad, _H, dh).transpose(1, 0, 2)
        k = qkv[:, c:2 * c].reshape(npad, _H, dh).transpose(1, 0, 2)
        v = qkv[:, 2 * c:].reshape(npad, _H, dh).transpose(1, 0, 2)
        o = _attention(q, k, v, batch_pad)
        o = o.transpose(1, 0, 2).reshape(npad, c)[:n]
        o = _linear(o, lp['wo'], lp['bo'])
        h2 = _bn(o + x0, lp['n2_g'], lp['n2_b'])

        out = h1 + h2
        ff = _linear(out, lp['ff_w1'], lp['ff_b1'], relu=True)
        ff = _linear(ff, lp['ff_w2'], lp['ff_b2'])
        x0 = _bn(out + ff, lp['n3_g'], lp['n3_b'])

    pooled = jax.ops.segment_sum(x0, batch, num_segments=_B)
    y = _linear(pooled, params['mlp_w1'], params['mlp_b1'], relu=True, bm=32)
    y = _linear(y, params['mlp_w2'], params['mlp_b2'], relu=True, bm=32)
    return _linear(y, params['mlp_w3'], params['mlp_b3'], bm=32)


# speedup vs baseline: 1.2884x; 1.0668x over previous
import functools
import math

import jax
import jax.numpy as jnp
from jax.experimental import pallas as pl
from jax.experimental.pallas import tpu as pltpu

_H = 4
_B = 20
_BQ = 256
_BK = 256
_PAD_BATCH = jnp.int32(1 << 20)


def _pad_rows(a, mult):
    n = a.shape[0]
    npad = -(-n // mult) * mult
    if npad == n:
        return a
    pad = [(0, npad - n)] + [(0, 0)] * (a.ndim - 1)
    return jnp.pad(a, pad)


def _linear_kernel(x_ref, w_ref, b_ref, o_ref, *, relu):
    y = jax.lax.dot_general(x_ref[...], w_ref[...], (((1,), (0,)), ((), ())),
                            preferred_element_type=jnp.float32)
    y = y + b_ref[...]
    if relu:
        y = jnp.maximum(y, 0.0)
    o_ref[...] = y


def _linear(x, w, b, relu=False, bm=256):
    n, k = x.shape
    kout = w.shape[1]
    xp = _pad_rows(x, bm)
    npad = xp.shape[0]
    out = pl.pallas_call(
        functools.partial(_linear_kernel, relu=relu),
        grid=(npad // bm,),
        in_specs=[
            pl.BlockSpec((bm, k), lambda i: (i, 0)),
            pl.BlockSpec((k, kout), lambda i: (0, 0)),
            pl.BlockSpec((1, kout), lambda i: (0, 0)),
        ],
        out_specs=pl.BlockSpec((bm, kout), lambda i: (i, 0)),
        out_shape=jax.ShapeDtypeStruct((npad, kout), jnp.float32),
    )(xp, w, b.reshape(1, -1))
    return out[:n]


def _bn_kernel(x_ref, g_ref, b_ref, o_ref, *, n_valid):
    x = x_ref[...]
    rows = jax.lax.broadcasted_iota(jnp.int32, x.shape, 0)
    valid = rows < n_valid
    xm = jnp.where(valid, x, 0.0)
    mu = jnp.sum(xm, axis=0, keepdims=True) / n_valid
    d = jnp.where(valid, x - mu, 0.0)
    var = jnp.sum(d * d, axis=0, keepdims=True) / n_valid
    o_ref[...] = (x - mu) / jnp.sqrt(var + 1e-5) * g_ref[...] + b_ref[...]


def _bn(x, g, b):
    n, c = x.shape
    xp = _pad_rows(x, 256)
    out = pl.pallas_call(
        functools.partial(_bn_kernel, n_valid=n),
        in_specs=[pl.BlockSpec(xp.shape, lambda: (0, 0)),
                  pl.BlockSpec((1, c), lambda: (0, 0)),
                  pl.BlockSpec((1, c), lambda: (0, 0))],
        out_specs=pl.BlockSpec(xp.shape, lambda: (0, 0)),
        out_shape=jax.ShapeDtypeStruct(xp.shape, jnp.float32),
    )(xp, g.reshape(1, -1), b.reshape(1, -1))
    return out[:n]


_EB = 256
_NB = 256


def _segsum_kernel(wlo_ref, whi_ref, xg_ref, ea_ref, dst_ref, x0_ref, o_ref):
    i = pl.program_id(0)

    @pl.when(i == 0)
    def _init():
        o_ref[...] = x0_ref[...]

    m = jnp.maximum(xg_ref[...] + ea_ref[...], 0.0)
    d = dst_ref[...]
    lo = wlo_ref[i]
    hi = whi_ref[i]

    def body(j, carry):
        node_base = j * _NB
        rowids = node_base + jax.lax.broadcasted_iota(jnp.int32, (_NB, _EB), 0)
        onehot = jnp.where(rowids == d, 1.0, 0.0)
        contrib = jax.lax.dot_general(onehot, m, (((1,), (0,)), ((), ())),
                                      preferred_element_type=jnp.float32)
        o_ref[pl.ds(node_base, _NB), :] = o_ref[pl.ds(node_base, _NB), :] + contrib
        return carry

    jax.lax.fori_loop(lo, hi + 1, body, 0)


def _gine_aggr(xg, ea_s, dst_sorted, x0):
    e, c = xg.shape
    n = x0.shape[0]
    xgp = _pad_rows(xg, _EB)
    eap = _pad_rows(ea_s, _EB)
    epad = xgp.shape[0]
    if epad != e:
        dst_sorted = jnp.concatenate(
            [dst_sorted, jnp.broadcast_to(dst_sorted[-1], (epad - e,))])
    npad = -(-n // _NB) * _NB
    x0p = _pad_rows(x0, _NB)
    num_eb = epad // _EB
    starts = jnp.arange(num_eb) * _EB
    wlo = (dst_sorted[starts] // _NB).astype(jnp.int32)
    whi = (dst_sorted[starts + _EB - 1] // _NB).astype(jnp.int32)

    grid_spec = pltpu.PrefetchScalarGridSpec(
        num_scalar_prefetch=2,
        grid=(num_eb,),
        in_specs=[
            pl.BlockSpec((_EB, c), lambda i, lo, hi: (i, 0)),
            pl.BlockSpec((_EB, c), lambda i, lo, hi: (i, 0)),
            pl.BlockSpec((1, _EB), lambda i, lo, hi: (0, i)),
            pl.BlockSpec((npad, c), lambda i, lo, hi: (0, 0)),
        ],
        out_specs=pl.BlockSpec((npad, c), lambda i, lo, hi: (0, 0)),
    )
    out = pl.pallas_call(
        _segsum_kernel,
        grid_spec=grid_spec,
        out_shape=jax.ShapeDtypeStruct((npad, c), jnp.float32),
    )(wlo, whi, xgp, eap, dst_sorted.reshape(1, -1).astype(jnp.int32), x0p)
    return out[:n]


def _attn_kernel(kblo_ref, kbhi_ref, q_ref, k_ref, v_ref, bcol_ref, brow_ref,
                 o_ref, *, scale, bk):
    qi = pl.program_id(1)
    lo = kblo_ref[qi]
    hi = kbhi_ref[qi]
    q = q_ref[...]
    bq = bcol_ref[...]

    def body(kb, carry):
        acc, m, l = carry
        kblk = k_ref[pl.ds(kb * bk, bk), :]
        vblk = v_ref[pl.ds(kb * bk, bk), :]
        bkey = brow_ref[:, pl.ds(kb * bk, bk)]
        s = jax.lax.dot_general(q, kblk, (((1,), (1,)), ((), ())),
                                preferred_element_type=jnp.float32) * scale
        mask = bq == bkey
        s = jnp.where(mask, s, -1e9)
        m_new = jnp.maximum(m, jnp.max(s, axis=1, keepdims=True))
        p = jnp.where(mask, jnp.exp(s - m_new), 0.0)
        alpha = jnp.exp(m - m_new)
        l_new = l * alpha + jnp.sum(p, axis=1, keepdims=True)
        acc_new = acc * alpha + jax.lax.dot_general(
            p, vblk, (((1,), (0,)), ((), ())), preferred_element_type=jnp.float32)
        return acc_new, m_new, l_new

    init = (jnp.zeros(q.shape, jnp.float32),
            jnp.full((q.shape[0], 1), -1e9, jnp.float32),
            jnp.zeros((q.shape[0], 1), jnp.float32))
    acc, m, l = jax.lax.fori_loop(lo, hi, body, init)
    l = jnp.where(l == 0.0, 1.0, l)
    o_ref[...] = acc / l


def _attention(q, k, v, batch_pad):
    h, npad, dh = q.shape
    num_qb = npad // _BQ
    scale = 1.0 / math.sqrt(dh)

    qstarts = jnp.arange(num_qb) * _BQ
    b_lo = batch_pad[qstarts]
    b_hi = batch_pad[qstarts + _BQ - 1]
    lo_idx = jnp.searchsorted(batch_pad, b_lo, side='left')
    hi_idx = jnp.searchsorted(batch_pad, b_hi, side='right')
    kb_lo = (lo_idx // _BK).astype(jnp.int32)
    kb_hi = ((hi_idx + _BK - 1) // _BK).astype(jnp.int32)

    bcol = batch_pad.reshape(-1, 1)
    brow = batch_pad.reshape(1, -1)

    grid_spec = pltpu.PrefetchScalarGridSpec(
        num_scalar_prefetch=2,
        grid=(h, num_qb),
        in_specs=[
            pl.BlockSpec((None, _BQ, dh), lambda hh, qi, lo, hi: (hh, qi, 0)),
            pl.BlockSpec((None, npad, dh), lambda hh, qi, lo, hi: (hh, 0, 0)),
            pl.BlockSpec((None, npad, dh), lambda hh, qi, lo, hi: (hh, 0, 0)),
            pl.BlockSpec((_BQ, 1), lambda hh, qi, lo, hi: (qi, 0)),
            pl.BlockSpec((1, npad), lambda hh, qi, lo, hi: (0, 0)),
        ],
        out_specs=pl.BlockSpec((None, _BQ, dh), lambda hh, qi, lo, hi: (hh, qi, 0)),
    )
    return pl.pallas_call(
        functools.partial(_attn_kernel, scale=scale, bk=_BK),
        grid_spec=grid_spec,
        out_shape=jax.ShapeDtypeStruct((h, npad, dh), jnp.float32),
    )(kb_lo, kb_hi, q, k, v, bcol, brow)


def kernel(x, pe, edge_index, edge_attr, batch, params):
    n, c = x.shape
    dh = c // _H
    src = edge_index[0]
    dst = edge_index[1]
    batch = batch.astype(jnp.int32)
    batch_pad = _pad_rows(batch, _BQ)
    npad = batch_pad.shape[0]
    if npad != n:
        batch_pad = batch_pad.at[n:].set(_PAD_BATCH)

    pe_mu = jnp.mean(pe, axis=0)
    pe_var = jnp.var(pe, axis=0)
    x_pe = (pe - pe_mu) / jnp.sqrt(pe_var + 1e-5) * params['pe_norm_g'] + params['pe_norm_b']
    pe_walk = pe.shape[1]
    c_node = params['node_emb_w'].shape[1]
    w_emb = jnp.zeros((c + pe_walk, c), jnp.float32)
    w_emb = w_emb.at[:c, :c_node].set(params['node_emb_w'])
    w_emb = w_emb.at[c:, c_node:].set(params['pe_lin_w'])
    b_emb = jnp.concatenate([params['node_emb_b'], params['pe_lin_b']])
    x0 = _linear(jnp.concatenate([x, x_pe], axis=1), w_emb, b_emb)

    ea = _linear(edge_attr, params['edge_emb_w'], params['edge_emb_b'])

    perm = jnp.argsort(dst)
    src_s = src[perm]
    dst_s = dst[perm]
    ea_s = ea[perm]

    for lp in params['layers']:
        hh = _gine_aggr(x0[src_s], ea_s, dst_s, x0)
        hh = _linear(hh, lp['gine_w1'], lp['gine_b1'], relu=True)
        hh = _linear(hh, lp['gine_w2'], lp['gine_b2'])
        h1 = _bn(hh + x0, lp['n1_g'], lp['n1_b'])

        w_qkv = jnp.concatenate([lp['wq'], lp['wk'], lp['wv']], axis=1)
        b_qkv = jnp.concatenate([lp['bq'], lp['bk'], lp['bv']])
        qkv = _pad_rows(_linear(x0, w_qkv, b_qkv), _BQ)
        q = qkv[:, :c].reshape(npad, _H, dh).transpose(1, 0, 2)
        k = qkv[:, c:2 * c].reshape(npad, _H, dh).transpose(1, 0, 2)
        v = qkv[:, 2 * c:].reshape(npad, _H, dh).transpose(1, 0, 2)
        o = _attention(q, k, v, batch_pad)
        o = o.transpose(1, 0, 2).reshape(npad, c)[:n]
        o = _linear(o, lp['wo'], lp['bo'])
        h2 = _bn(o + x0, lp['n2_g'], lp['n2_b'])

        out = h1 + h2
        ff = _linear(out, lp['ff_w1'], lp['ff_b1'], relu=True)
        ff = _linear(ff, lp['ff_w2'], lp['ff_b2'])
        x0 = _bn(out + ff, lp['n3_g'], lp['n3_b'])

    pooled = jax.ops.segment_sum(x0, batch, num_segments=_B)
    y = _linear(pooled, params['mlp_w1'], params['mlp_b1'], relu=True, bm=32)
    y = _linear(y, params['mlp_w2'], params['mlp_b2'], relu=True, bm=32)
    return _linear(y, params['mlp_w3'], params['mlp_b3'], bm=32)
